# baseline (speedup 1.0000x reference)
import functools

import jax
import jax.numpy as jnp
from jax import lax
from jax.experimental import pallas as pl
from jax.experimental.pallas import tpu as pltpu
from jax.experimental.pallas import tpu_sc as plsc

_BATCH = 16384
_D = 64
_NK = _D // 16
_CHUNK = 128


_GATHER_DNUMS = lax.GatherDimensionNumbers(
    offset_dims=(), collapsed_slice_dims=(0,), start_index_map=(0,))


def _lane_perm(x, idx):
  return lax.gather(x, idx[:, None], _GATHER_DNUMS, (1,),
                    mode=lax.GatherScatterMode.PROMISE_IN_BOUNDS)


def _lane_sum(x, lane):
  for s in (1, 2, 4, 8):
    x = x + _lane_perm(x, lane ^ s)
  return x


def _vec_sqrt(ss):
  i = lax.bitcast_convert_type(ss, jnp.int32)
  i = jnp.full((16,), 0x5F3759DF, dtype=jnp.int32) - (i >> 1)
  y = lax.bitcast_convert_type(i, jnp.float32)
  half_ss = ss * 0.5
  for _ in range(3):
    y = y * (1.5 - half_ss * y * y)
  return ss * y


def _transh_body(nw, bpw, n_chunks,
                 head_h, rel_h, tail_h, ent_h, relt_h, normt_h, out_h,
                 hidx, tidx, ridx, her, ter, rer, nnr, outv, sem):
  num_cores = plsc.get_sparse_core_info().num_cores
  wid = lax.axis_index("s") * num_cores + lax.axis_index("c")
  base = wid * bpw

  idx_copies = []
  for c in range(n_chunks):
    off = base + c * _CHUNK
    idx_copies.append(
        pltpu.async_copy(head_h.at[pl.ds(off, _CHUNK)], hidx.at[c], sem))
    idx_copies.append(
        pltpu.async_copy(tail_h.at[pl.ds(off, _CHUNK)], tidx.at[c], sem))
    idx_copies.append(
        pltpu.async_copy(rel_h.at[pl.ds(off, _CHUNK)], ridx.at[c], sem))
  for cp in idx_copies:
    cp.wait()

  lane = lax.iota(jnp.int32, 16)

  def chunk_body(c, carry):
    gathers = [
        pltpu.async_copy(ent_h.at[hidx.at[c]], her, sem),
        pltpu.async_copy(ent_h.at[tidx.at[c]], ter, sem),
        pltpu.async_copy(relt_h.at[ridx.at[c]], rer, sem),
        pltpu.async_copy(normt_h.at[ridx.at[c]], nnr, sem),
    ]
    for cp in gathers:
      cp.wait()

    def group_body(g, carry2):
      out_acc = jnp.zeros((16,), jnp.float32)
      e0 = g * 16
      for j in range(16):
        e = e0 + j
        h = [her[e, pl.ds(16 * k, 16)] for k in range(_NK)]
        t = [ter[e, pl.ds(16 * k, 16)] for k in range(_NK)]
        r = [rer[e, pl.ds(16 * k, 16)] for k in range(_NK)]
        n = [nnr[e, pl.ds(16 * k, 16)] for k in range(_NK)]
        d = [h[k] - t[k] for k in range(_NK)]
        dotv = d[0] * n[0]
        s2v = n[0] * n[0]
        for k in range(1, _NK):
          dotv = dotv + d[k] * n[k]
          s2v = s2v + n[k] * n[k]
        coeff = _lane_sum(dotv, lane) / _lane_sum(s2v, lane)
        acc = None
        for k in range(_NK):
          dv = d[k] + r[k] - coeff * n[k]
          acc = dv * dv if acc is None else acc + dv * dv
        out_acc = jnp.where(lane == j, _lane_sum(acc, lane), out_acc)
      outv[pl.ds(c * _CHUNK + e0, 16)] = _vec_sqrt(out_acc)
      return carry2

    return lax.fori_loop(0, _CHUNK // 16, group_body, carry)

  lax.fori_loop(0, n_chunks, chunk_body, 0)
  pltpu.sync_copy(outv, out_h.at[pl.ds(base, bpw)])


def kernel(head, relation, tail, entity_table, relation_table, norm_table):
  info = plsc.get_sparse_core_info()
  nw = info.num_cores * info.num_subcores
  bpw = _BATCH // nw
  n_chunks = bpw // _CHUNK
  mesh = plsc.VectorSubcoreMesh(core_axis_name="c", subcore_axis_name="s")

  transh = functools.partial(_transh_body, nw, bpw, n_chunks)
  run = pl.kernel(
      transh,
      out_type=jax.ShapeDtypeStruct((_BATCH,), jnp.float32),
      mesh=mesh,
      compiler_params=pltpu.CompilerParams(use_tc_tiling_on_sc=False),
      scratch_types=[
          pltpu.VMEM((n_chunks, _CHUNK), jnp.int32),
          pltpu.VMEM((n_chunks, _CHUNK), jnp.int32),
          pltpu.VMEM((n_chunks, _CHUNK), jnp.int32),
          pltpu.VMEM((_CHUNK, _D), jnp.float32),
          pltpu.VMEM((_CHUNK, _D), jnp.float32),
          pltpu.VMEM((_CHUNK, _D), jnp.float32),
          pltpu.VMEM((_CHUNK, _D), jnp.float32),
          pltpu.VMEM((bpw,), jnp.float32),
          pltpu.SemaphoreType.DMA,
      ],
  )
  return run(head, relation, tail, entity_table, relation_table, norm_table)


# speedup vs baseline: 1.5917x; 1.5917x over previous
import functools

import jax
import jax.numpy as jnp
from jax import lax
from jax.experimental import pallas as pl
from jax.experimental.pallas import tpu as pltpu
from jax.experimental.pallas import tpu_sc as plsc

_BATCH = 16384
_NREL = 1000
_D = 64
_NK = _D // 16
_CHUNK = 128

_GATHER_DNUMS = lax.GatherDimensionNumbers(
    offset_dims=(), collapsed_slice_dims=(0,), start_index_map=(0,))


def _lane_perm(x, idx):
  return lax.gather(x, idx[:, None], _GATHER_DNUMS, (1,),
                    mode=lax.GatherScatterMode.PROMISE_IN_BOUNDS)


def _lane_sum(x, lane):
  for s in (1, 2, 4, 8):
    x = x + _lane_perm(x, lane ^ s)
  return x


def _vec_sqrt(ss):
  i = lax.bitcast_convert_type(ss, jnp.int32)
  i = jnp.full((16,), 0x5F3759DF, dtype=jnp.int32) - (i >> 1)
  y = lax.bitcast_convert_type(i, jnp.float32)
  half_ss = ss * 0.5
  for _ in range(3):
    y = y * (1.5 - half_ss * y * y)
  return ss * y


def _transh_body(nw, bpw, n_chunks,
                 head_h, rel_h, tail_h, ent_h, relt_h, normt_h, out_h,
                 hidx, tidx, ridx, her, ter, rer, nnr, outv, sem):
  num_cores = plsc.get_sparse_core_info().num_cores
  sid = lax.axis_index("s")
  wid = sid * num_cores + lax.axis_index("c")
  base = wid * bpw

  cp_h = pltpu.async_copy(head_h.at[pl.ds(base, bpw)], hidx, sem)
  cp_t = pltpu.async_copy(tail_h.at[pl.ds(base, bpw)], tidx, sem)
  cp_r = pltpu.async_copy(rel_h.at[pl.ds(base, bpw)], ridx, sem)
  cp_h.wait()
  cp_t.wait()
  cp_r.wait()

  lane = lax.iota(jnp.int32, 16)

  def chunk_body(c, carry):
    c0 = c * _CHUNK

    def fire_body(g, carry2):
      e0 = g * 16
      hv = hidx[pl.ds(c0 + e0, 16)]
      tv = tidx[pl.ds(c0 + e0, 16)]
      rv = ridx[pl.ds(c0 + e0, 16)]
      for j in range(16):
        e = e0 + j
        pltpu.async_copy(ent_h.at[hv[j]], her.at[e], sem)
        pltpu.async_copy(ent_h.at[tv[j]], ter.at[e], sem)
        pltpu.async_copy(relt_h.at[rv[j]], rer.at[e], sem)
        pltpu.async_copy(normt_h.at[rv[j]], nnr.at[e], sem)
      pltpu.make_async_copy(
          ent_h.at[pl.ds(0, 16)], her.at[pl.ds(e0, 16)], sem).wait()
      pltpu.make_async_copy(
          ent_h.at[pl.ds(0, 16)], ter.at[pl.ds(e0, 16)], sem).wait()
      pltpu.make_async_copy(
          relt_h.at[pl.ds(0, 16)], rer.at[pl.ds(e0, 16)], sem).wait()
      pltpu.make_async_copy(
          normt_h.at[pl.ds(0, 16)], nnr.at[pl.ds(e0, 16)], sem).wait()
      return carry2

    lax.fori_loop(0, _CHUNK // 16, fire_body, 0)

    def group_body(g, carry2):
      out_acc = jnp.zeros((16,), jnp.float32)
      e0 = g * 16
      for j in range(16):
        e = e0 + j
        h = [her[e, pl.ds(16 * k, 16)] for k in range(_NK)]
        t = [ter[e, pl.ds(16 * k, 16)] for k in range(_NK)]
        r = [rer[e, pl.ds(16 * k, 16)] for k in range(_NK)]
        n = [nnr[e, pl.ds(16 * k, 16)] for k in range(_NK)]
        d = [h[k] - t[k] for k in range(_NK)]
        dotv = d[0] * n[0]
        s2v = n[0] * n[0]
        for k in range(1, _NK):
          dotv = dotv + d[k] * n[k]
          s2v = s2v + n[k] * n[k]
        coeff = _lane_sum(dotv, lane) / _lane_sum(s2v, lane)
        acc = None
        for k in range(_NK):
          dv = d[k] + r[k] - coeff * n[k]
          acc = dv * dv if acc is None else acc + dv * dv
        out_acc = jnp.where(lane == j, _lane_sum(acc, lane), out_acc)
      outv[pl.ds(c * _CHUNK + e0, 16)] = _vec_sqrt(out_acc)
      return carry2

    return lax.fori_loop(0, _CHUNK // 16, group_body, carry)

  lax.fori_loop(0, n_chunks, chunk_body, 0)
  pltpu.sync_copy(outv, out_h.at[pl.ds(base, bpw)])


def kernel(head, relation, tail, entity_table, relation_table, norm_table):
  info = plsc.get_sparse_core_info()
  nw = info.num_cores * info.num_subcores
  bpw = _BATCH // nw
  n_chunks = bpw // _CHUNK
  mesh = plsc.VectorSubcoreMesh(core_axis_name="c", subcore_axis_name="s")

  transh = functools.partial(_transh_body, nw, bpw, n_chunks)
  run = pl.kernel(
      transh,
      out_type=jax.ShapeDtypeStruct((_BATCH,), jnp.float32),
      mesh=mesh,
      scratch_types=[
          pltpu.VMEM((bpw,), jnp.int32),
          pltpu.VMEM((bpw,), jnp.int32),
          pltpu.VMEM((bpw,), jnp.int32),
          pltpu.VMEM((_CHUNK, _D), jnp.float32),
          pltpu.VMEM((_CHUNK, _D), jnp.float32),
          pltpu.VMEM((_CHUNK, _D), jnp.float32),
          pltpu.VMEM((_CHUNK, _D), jnp.float32),
          pltpu.VMEM((bpw,), jnp.float32),
          pltpu.SemaphoreType.DMA,
      ],
  )
  return run(head, relation, tail, entity_table, relation_table, norm_table)


# speedup vs baseline: 1.6628x; 1.0446x over previous
import functools

import jax
import jax.numpy as jnp
from jax import lax
from jax.experimental import pallas as pl
from jax.experimental.pallas import tpu as pltpu
from jax.experimental.pallas import tpu_sc as plsc

_BATCH = 16384
_NREL = 1000
_D = 64
_NK = _D // 16
_G = 16

_GATHER_DNUMS = lax.GatherDimensionNumbers(
    offset_dims=(), collapsed_slice_dims=(0,), start_index_map=(0,))


def _lane_perm(x, idx):
  return lax.gather(x, idx[:, None], _GATHER_DNUMS, (1,),
                    mode=lax.GatherScatterMode.PROMISE_IN_BOUNDS)


def _lane_sum(x, lane):
  for s in (1, 2, 4, 8):
    x = x + _lane_perm(x, lane ^ s)
  return x


def _vec_sqrt(ss):
  i = lax.bitcast_convert_type(ss, jnp.int32)
  i = jnp.full((16,), 0x5F3759DF, dtype=jnp.int32) - (i >> 1)
  y = lax.bitcast_convert_type(i, jnp.float32)
  half_ss = ss * 0.5
  for _ in range(3):
    y = y * (1.5 - half_ss * y * y)
  return ss * y


def _transh_body(nw, bpw,
                 head_h, rel_h, tail_h, ent_h, relt_h, normt_h, out_h,
                 hidx, tidx, ridx, her, ter, rer, nnr, outv, sem0, sem1):
  num_cores = plsc.get_sparse_core_info().num_cores
  wid = lax.axis_index("s") * num_cores + lax.axis_index("c")
  base = wid * bpw
  n_groups = bpw // _G

  cp_h = pltpu.async_copy(head_h.at[pl.ds(base, bpw)], hidx, sem0)
  cp_t = pltpu.async_copy(tail_h.at[pl.ds(base, bpw)], tidx, sem0)
  cp_r = pltpu.async_copy(rel_h.at[pl.ds(base, bpw)], ridx, sem0)
  cp_h.wait()
  cp_t.wait()
  cp_r.wait()

  lane = lax.iota(jnp.int32, 16)
  sems = (sem0, sem1)

  def fire(g, p, sem):
    e0 = g * _G
    hv = hidx[pl.ds(e0, 16)]
    tv = tidx[pl.ds(e0, 16)]
    rv = ridx[pl.ds(e0, 16)]
    for j in range(16):
      pltpu.async_copy(ent_h.at[hv[j]], her.at[p, j], sem)
      pltpu.async_copy(ent_h.at[tv[j]], ter.at[p, j], sem)
      pltpu.async_copy(relt_h.at[rv[j]], rer.at[p, j], sem)
      pltpu.async_copy(normt_h.at[rv[j]], nnr.at[p, j], sem)

  def drain(p, sem):
    pltpu.make_async_copy(ent_h.at[pl.ds(0, 16)], her.at[p], sem).wait()
    pltpu.make_async_copy(ent_h.at[pl.ds(0, 16)], ter.at[p], sem).wait()
    pltpu.make_async_copy(relt_h.at[pl.ds(0, 16)], rer.at[p], sem).wait()
    pltpu.make_async_copy(normt_h.at[pl.ds(0, 16)], nnr.at[p], sem).wait()

  def compute(g, p):
    out_acc = jnp.zeros((16,), jnp.float32)
    for j in range(16):
      h = [her[p, j, pl.ds(16 * k, 16)] for k in range(_NK)]
      t = [ter[p, j, pl.ds(16 * k, 16)] for k in range(_NK)]
      r = [rer[p, j, pl.ds(16 * k, 16)] for k in range(_NK)]
      n = [nnr[p, j, pl.ds(16 * k, 16)] for k in range(_NK)]
      d = [h[k] - t[k] for k in range(_NK)]
      dotv = d[0] * n[0]
      s2v = n[0] * n[0]
      for k in range(1, _NK):
        dotv = dotv + d[k] * n[k]
        s2v = s2v + n[k] * n[k]
      coeff = _lane_sum(dotv, lane) / _lane_sum(s2v, lane)
      acc = None
      for k in range(_NK):
        dv = d[k] + r[k] - coeff * n[k]
        acc = dv * dv if acc is None else acc + dv * dv
      out_acc = jnp.where(lane == j, _lane_sum(acc, lane), out_acc)
    outv[pl.ds(g * _G, 16)] = _vec_sqrt(out_acc)

  fire(0, 0, sem0)

  def pair_body(gg, carry):
    g = gg * 2

    @pl.when(g + 1 < n_groups)
    def _():
      fire(g + 1, 1, sem1)

    drain(0, sem0)
    compute(g, 0)

    @pl.when(g + 2 < n_groups)
    def _():
      fire(g + 2, 0, sem0)

    @pl.when(g + 1 < n_groups)
    def _():
      drain(1, sem1)
      compute(g + 1, 1)

    return carry

  lax.fori_loop(0, (n_groups + 1) // 2, pair_body, 0)
  pltpu.sync_copy(outv, out_h.at[pl.ds(base, bpw)])


def kernel(head, relation, tail, entity_table, relation_table, norm_table):
  info = plsc.get_sparse_core_info()
  nw = info.num_cores * info.num_subcores
  bpw = _BATCH // nw
  mesh = plsc.VectorSubcoreMesh(core_axis_name="c", subcore_axis_name="s")

  transh = functools.partial(_transh_body, nw, bpw)
  run = pl.kernel(
      transh,
      out_type=jax.ShapeDtypeStruct((_BATCH,), jnp.float32),
      mesh=mesh,
      scratch_types=[
          pltpu.VMEM((bpw,), jnp.int32),
          pltpu.VMEM((bpw,), jnp.int32),
          pltpu.VMEM((bpw,), jnp.int32),
          pltpu.VMEM((2, _G, _D), jnp.float32),
          pltpu.VMEM((2, _G, _D), jnp.float32),
          pltpu.VMEM((2, _G, _D), jnp.float32),
          pltpu.VMEM((2, _G, _D), jnp.float32),
          pltpu.VMEM((bpw,), jnp.float32),
          pltpu.SemaphoreType.DMA,
          pltpu.SemaphoreType.DMA,
      ],
  )
  return run(head, relation, tail, entity_table, relation_table, norm_table)


# speedup vs baseline: 1.6685x; 1.0035x over previous
import functools

import jax
import jax.numpy as jnp
from jax import lax
from jax.experimental import pallas as pl
from jax.experimental.pallas import tpu as pltpu
from jax.experimental.pallas import tpu_sc as plsc

_BATCH = 16384
_NREL = 1000
_D = 64
_NK = _D // 16
_G = 16

_GATHER_DNUMS = lax.GatherDimensionNumbers(
    offset_dims=(), collapsed_slice_dims=(0,), start_index_map=(0,))


def _lane_perm(x, idx):
  return lax.gather(x, idx[:, None], _GATHER_DNUMS, (1,),
                    mode=lax.GatherScatterMode.PROMISE_IN_BOUNDS)


def _lane_sum(x, lane):
  for s in (1, 2, 4, 8):
    x = x + _lane_perm(x, lane ^ s)
  return x


def _vec_sqrt(ss):
  i = lax.bitcast_convert_type(ss, jnp.int32)
  i = jnp.full((16,), 0x5F3759DF, dtype=jnp.int32) - (i >> 1)
  y = lax.bitcast_convert_type(i, jnp.float32)
  half_ss = ss * 0.5
  for _ in range(3):
    y = y * (1.5 - half_ss * y * y)
  return ss * y


def _transh_body(nw, bpw,
                 head_h, rel_h, tail_h, ent_h, relt_h, normt_h, out_h,
                 hidx, tidx, ridx, her, ter, rnr, outv, sem0, sem1):
  num_cores = plsc.get_sparse_core_info().num_cores
  wid = lax.axis_index("s") * num_cores + lax.axis_index("c")
  base = wid * bpw
  n_groups = bpw // _G

  cp_h = pltpu.async_copy(head_h.at[pl.ds(base, bpw)], hidx, sem0)
  cp_t = pltpu.async_copy(tail_h.at[pl.ds(base, bpw)], tidx, sem0)
  cp_r = pltpu.async_copy(rel_h.at[pl.ds(base, bpw)], ridx, sem0)
  cp_h.wait()
  cp_t.wait()
  cp_r.wait()

  lane = lax.iota(jnp.int32, 16)
  sems = (sem0, sem1)

  def fire(g, p, sem):
    e0 = g * _G
    hv = hidx[pl.ds(e0, 16)]
    tv = tidx[pl.ds(e0, 16)]
    rv = ridx[pl.ds(e0, 16)]
    for j in range(16):
      pltpu.async_copy(ent_h.at[hv[j]], her.at[p, j], sem)
      pltpu.async_copy(ent_h.at[tv[j]], ter.at[p, j], sem)
      pltpu.async_copy(relt_h.at[rv[j]], rnr.at[p, j], sem)

  def drain(p, sem):
    pltpu.make_async_copy(ent_h.at[pl.ds(0, 16)], her.at[p], sem).wait()
    pltpu.make_async_copy(ent_h.at[pl.ds(0, 16)], ter.at[p], sem).wait()
    pltpu.make_async_copy(relt_h.at[pl.ds(0, 16)], rnr.at[p], sem).wait()

  def compute(g, p):
    out_acc = jnp.zeros((16,), jnp.float32)
    for j in range(16):
      h = [her[p, j, pl.ds(16 * k, 16)] for k in range(_NK)]
      t = [ter[p, j, pl.ds(16 * k, 16)] for k in range(_NK)]
      r = [rnr[p, j, pl.ds(16 * k, 16)] for k in range(_NK)]
      n = [rnr[p, j, pl.ds(_D + 16 * k, 16)] for k in range(_NK)]
      d = [h[k] - t[k] for k in range(_NK)]
      dotv = d[0] * n[0]
      s2v = n[0] * n[0]
      for k in range(1, _NK):
        dotv = dotv + d[k] * n[k]
        s2v = s2v + n[k] * n[k]
      coeff = _lane_sum(dotv, lane) / _lane_sum(s2v, lane)
      acc = None
      for k in range(_NK):
        dv = d[k] + r[k] - coeff * n[k]
        acc = dv * dv if acc is None else acc + dv * dv
      out_acc = jnp.where(lane == j, _lane_sum(acc, lane), out_acc)
    outv[pl.ds(g * _G, 16)] = _vec_sqrt(out_acc)

  fire(0, 0, sem0)

  def pair_body(gg, carry):
    g = gg * 2

    @pl.when(g + 1 < n_groups)
    def _():
      fire(g + 1, 1, sem1)

    drain(0, sem0)
    compute(g, 0)

    @pl.when(g + 2 < n_groups)
    def _():
      fire(g + 2, 0, sem0)

    @pl.when(g + 1 < n_groups)
    def _():
      drain(1, sem1)
      compute(g + 1, 1)

    return carry

  lax.fori_loop(0, (n_groups + 1) // 2, pair_body, 0)
  pltpu.sync_copy(outv, out_h.at[pl.ds(base, bpw)])


def kernel(head, relation, tail, entity_table, relation_table, norm_table):
  info = plsc.get_sparse_core_info()
  nw = info.num_cores * info.num_subcores
  bpw = _BATCH // nw
  mesh = plsc.VectorSubcoreMesh(core_axis_name="c", subcore_axis_name="s")

  transh = functools.partial(_transh_body, nw, bpw)
  run = pl.kernel(
      transh,
      out_type=jax.ShapeDtypeStruct((_BATCH,), jnp.float32),
      mesh=mesh,
      scratch_types=[
          pltpu.VMEM((bpw,), jnp.int32),
          pltpu.VMEM((bpw,), jnp.int32),
          pltpu.VMEM((bpw,), jnp.int32),
          pltpu.VMEM((2, _G, _D), jnp.float32),
          pltpu.VMEM((2, _G, _D), jnp.float32),
          pltpu.VMEM((2, _G, 2 * _D), jnp.float32),
          pltpu.VMEM((bpw,), jnp.float32),
          pltpu.SemaphoreType.DMA,
          pltpu.SemaphoreType.DMA,
      ],
  )
  rel_cat = jnp.concatenate([relation_table, norm_table], axis=1)
  return run(head, relation, tail, entity_table, rel_cat, norm_table)
